# baseline (speedup 1.0000x reference)
import jax
import jax.numpy as jnp
from jax import lax
from jax.experimental import pallas as pl
from jax.experimental.pallas import tpu as pltpu
from jax.experimental.pallas import tpu_sc as plsc

_B = 1024
_S = 200
_D = 64
_NC = 2
_NS = 16
_NW = _NC * _NS
_UPW = _B // _NW
_L = 16
_VPR = _D // _L


def _emb_body(x_hbm, tab_hbm, pe_hbm, out_hbm,
              xbuf, pe_v, gb0, gb1, gsem0, gsem1, ssem0, ssem1):
    wid = lax.axis_index("s") * _NC + lax.axis_index("c")
    b_lo = wid * _UPW

    pltpu.sync_copy(pe_hbm, pe_v)
    pltpu.sync_copy(x_hbm.at[pl.ds(b_lo, _UPW)], xbuf)

    gb = (gb0, gb1)
    gsems = (gsem0, gsem1)
    ssems = (ssem0, ssem1)

    def fire(u, k):
        pltpu.make_async_copy(
            tab_hbm.at[xbuf.at[u]], gb[k], gsems[k]).start()

    def wait_gather(u, k):
        pltpu.make_async_copy(
            tab_hbm.at[xbuf.at[u]], gb[k], gsems[k]).wait()

    def add_pe(k):
        g_ = gb[k]

        def r_body(r, carry):
            for j in range(_VPR):
                sl = pl.ds(j * _L, _L)
                g_[r, sl] = g_[r, sl] + pe_v[r, sl]
            return carry

        lax.fori_loop(0, _S, r_body, 0, unroll=4)

    def store_cp(u, k):
        return pltpu.make_async_copy(
            gb[k], out_hbm.at[pl.ds((b_lo + u) * _S, _S)], ssems[k])

    fire(0, 0)
    wait_gather(0, 0)
    fire(1, 1)
    add_pe(0)
    store_cp(0, 0).start()
    wait_gather(1, 1)
    store_cp(0, 0).wait()
    fire(2, 0)
    add_pe(1)
    store_cp(1, 1).start()

    def pair_body(p, carry):
        for k in range(2):
            u = 2 * p + k
            wait_gather(u, k)
            store_cp(u - 1, 1 - k).wait()
            fire(u + 1, 1 - k)
            add_pe(k)
            store_cp(u, k).start()
        return carry

    lax.fori_loop(1, _UPW // 2 - 1, pair_body, 0)

    wait_gather(_UPW - 2, 0)
    store_cp(_UPW - 3, 1).wait()
    fire(_UPW - 1, 1)
    add_pe(0)
    store_cp(_UPW - 2, 0).start()
    wait_gather(_UPW - 1, 1)
    store_cp(_UPW - 2, 0).wait()
    add_pe(1)
    store_cp(_UPW - 1, 1).start()
    store_cp(_UPW - 1, 1).wait()


def _emb_call(x, table, pe):
    mesh = plsc.VectorSubcoreMesh(
        core_axis_name="c", subcore_axis_name="s",
        num_cores=_NC, num_subcores=_NS)
    return pl.kernel(
        _emb_body,
        out_type=jax.ShapeDtypeStruct((_B * _S, _D), jnp.float32),
        mesh=mesh,
        compiler_params=pltpu.CompilerParams(use_tc_tiling_on_sc=False),
        scratch_types=[
            pltpu.VMEM((_UPW, _S), jnp.int32),
            pltpu.VMEM((_S, _D), jnp.float32),
            pltpu.VMEM((_S, _D), jnp.float32),
            pltpu.VMEM((_S, _D), jnp.float32),
            pltpu.SemaphoreType.DMA,
            pltpu.SemaphoreType.DMA,
            pltpu.SemaphoreType.DMA,
            pltpu.SemaphoreType.DMA,
        ],
    )(x, table, pe)


def kernel(x, table, pe):
    pe_s = pe[: x.shape[1]]
    out_flat = _emb_call(x.astype(jnp.int32), table, pe_s)
    return out_flat.reshape(x.shape[0], x.shape[1], _D)
